# baseline (speedup 1.0000x reference)
import functools

import jax
import jax.numpy as jnp
from jax import lax
from jax.experimental import pallas as pl
from jax.experimental.pallas import tpu as pltpu
from jax.experimental.pallas import tpu_sc as plsc

_info = plsc.get_sparse_core_info()
_NC, _NS, _L = _info.num_cores, _info.num_subcores, _info.num_lanes
_NW = _NC * _NS

_BATCH, _SEQ, _VOCAB, _DIM = 4, 8192, 8, 4
_TOK = _BATCH * _SEQ
_TILES = _SEQ // 128
_TPW = _TOK // _NW
_OPW = _TPW * _DIM
_SEG = _DIM * 128


@functools.partial(
    pl.kernel,
    mesh=plsc.VectorSubcoreMesh(core_axis_name="c", subcore_axis_name="s"),
    out_type=jax.ShapeDtypeStruct((_TOK * _DIM,), jnp.float32),
    compiler_params=pltpu.CompilerParams(
        needs_layout_passes=False, use_tc_tiling_on_sc=False
    ),
    scratch_types=[
        pltpu.VMEM((2, 512), jnp.int32),
        pltpu.VMEM((_VOCAB * _DIM,), jnp.float32),
        pltpu.VMEM((_OPW,), jnp.float32),
        pltpu.SemaphoreType.DMA,
        pltpu.SemaphoreType.DMA,
        pltpu.SemaphoreType.DMA,
    ],
)
def _embed_sc(ids_hbm, tab_hbm, out_hbm, ids_v, tab_v, out_v, sem_t, sem_i, sem_o):
    wid = lax.axis_index("s") * _NC + lax.axis_index("c")
    cp_t = pltpu.async_copy(tab_hbm, tab_v, sem_t)
    cp_i = pltpu.async_copy(ids_hbm.at[pl.ds(2 * wid, 2)], ids_v, sem_i)
    cp_t.wait()
    cp_i.wait()

    def seg(j, carry):
        s_hi_l = j // _BATCH
        b = j % _BATCH
        for s16 in range(8):
            ids16 = ids_v[s_hi_l, pl.ds(b * 128 + s16 * _L, _L)]
            for d in range(_DIM):
                col = plsc.load_gather(tab_v, [ids16 + _VOCAB * d])
                out_v[pl.ds(j * _SEG + d * 128 + s16 * _L, _L)] = col
        dst = b * (_TILES * _SEG) + (2 * wid + s_hi_l) * _SEG
        pltpu.async_copy(
            out_v.at[pl.ds(j * _SEG, _SEG)], out_hbm.at[pl.ds(dst, _SEG)], sem_o
        )
        return carry

    lax.fori_loop(0, _OPW // _SEG, seg, 0)
    pltpu.make_async_copy(out_v, out_hbm.at[pl.ds(wid * _OPW, _OPW)], sem_o).wait()


def kernel(input_ids, embed_weight):
    ids = input_ids.reshape(_BATCH, _TILES, 128).transpose(1, 0, 2)
    ids = ids.reshape(_TILES, _BATCH * 128)
    tab = embed_weight.T.reshape(-1)
    out = _embed_sc(ids, tab)
    out = out.reshape(_BATCH, _TILES, _DIM, 128).transpose(0, 1, 3, 2)
    return out.reshape(_BATCH, _SEQ, _DIM)


# speedup vs baseline: 1.0052x; 1.0052x over previous
import functools

import jax
import jax.numpy as jnp
from jax import lax
from jax.experimental import pallas as pl
from jax.experimental.pallas import tpu as pltpu
from jax.experimental.pallas import tpu_sc as plsc

_info = plsc.get_sparse_core_info()
_NC, _NS, _L = _info.num_cores, _info.num_subcores, _info.num_lanes
_NW = _NC * _NS

_BATCH, _SEQ, _VOCAB, _DIM = 4, 8192, 8, 4
_TOK = _BATCH * _SEQ
_TILES = _SEQ // 128
_GROUPS = _TILES // 8
_TPW = _TOK // _NW
_OPW = _TPW * _DIM
_STEPS = _TPW // _L


@functools.partial(
    pl.kernel,
    mesh=plsc.VectorSubcoreMesh(core_axis_name="c", subcore_axis_name="s"),
    out_type=jax.ShapeDtypeStruct((_TOK * _DIM,), jnp.float32),
    compiler_params=pltpu.CompilerParams(
        needs_layout_passes=False,
        use_tc_tiling_on_sc=False,
        disable_bounds_checks=True,
    ),
    scratch_types=[
        pltpu.VMEM((8, 128), jnp.int32),
        pltpu.VMEM((_VOCAB * _DIM,), jnp.float32),
        pltpu.VMEM((_OPW,), jnp.float32),
        pltpu.SemaphoreType.DMA,
        pltpu.SemaphoreType.DMA,
    ],
)
def _embed_sc(ids_hbm, tab_hbm, out_hbm, ids_v, tab_v, out_v, sem_t, sem_i):
    wid = lax.axis_index("s") * _NC + lax.axis_index("c")
    b = wid // _GROUPS
    g = wid % _GROUPS
    cp_t = pltpu.async_copy(tab_hbm, tab_v, sem_t)
    cp_i = pltpu.async_copy(
        ids_hbm.at[pl.ds(g * 8, 8), pl.ds(b * 128, 128)], ids_v, sem_i
    )
    cp_t.wait()
    cp_i.wait()

    def step(t, carry):
        s_hi = t // 8
        s16 = t % 8
        ids16 = ids_v[s_hi, pl.ds(s16 * _L, _L)]
        off = s_hi * 512 + s16 * _L
        for d in range(_DIM):
            col = plsc.load_gather(tab_v, [ids16 + _VOCAB * d])
            out_v[pl.ds(off + d * 128, _L)] = col
        return carry

    lax.fori_loop(0, _STEPS, step, 0, unroll=4)
    pltpu.sync_copy(out_v, out_hbm.at[pl.ds(wid * _OPW, _OPW)])


def kernel(input_ids, embed_weight):
    ids = input_ids.reshape(_BATCH, _TILES, 128).transpose(1, 0, 2)
    ids = ids.reshape(_TILES, _BATCH * 128)
    tab = embed_weight.T.reshape(-1)
    out = _embed_sc(ids, tab)
    out = out.reshape(_BATCH, _TILES, _DIM, 128).transpose(0, 1, 3, 2)
    return out.reshape(_BATCH, _SEQ, _DIM)


# speedup vs baseline: 1.0086x; 1.0033x over previous
import functools

import jax
import jax.numpy as jnp
from jax import lax
from jax.experimental import pallas as pl
from jax.experimental.pallas import tpu as pltpu
from jax.experimental.pallas import tpu_sc as plsc

_info = plsc.get_sparse_core_info()
_NC, _NS, _L = _info.num_cores, _info.num_subcores, _info.num_lanes
_NW = _NC * _NS

_BATCH, _SEQ, _VOCAB, _DIM = 4, 8192, 8, 4
_TOK = _BATCH * _SEQ
_TILES = _SEQ // 128
_GROUPS = _TILES // 8
_TPW = _TOK // _NW
_OPW = _TPW * _DIM
_STEPS = _TPW // _L
_TAB = _VOCAB * _DIM


@functools.partial(
    pl.kernel,
    mesh=plsc.VectorSubcoreMesh(core_axis_name="c", subcore_axis_name="s"),
    out_type=jax.ShapeDtypeStruct((_TOK * _DIM,), jnp.float32),
    compiler_params=pltpu.CompilerParams(
        needs_layout_passes=False,
        use_tc_tiling_on_sc=False,
        disable_bounds_checks=True,
    ),
    scratch_types=[
        pltpu.VMEM((8, 128), jnp.int32),
        pltpu.VMEM((_TAB,), jnp.int32),
        pltpu.VMEM((_OPW,), jnp.float32),
        pltpu.SemaphoreType.DMA,
        pltpu.SemaphoreType.DMA,
    ],
)
def _embed_sc(in_hbm, out_hbm, ids_v, tab_v, out_v, sem_t, sem_i):
    wid = lax.axis_index("s") * _NC + lax.axis_index("c")
    b = wid // _GROUPS
    g = wid % _GROUPS
    cp_t = pltpu.async_copy(in_hbm.at[_TILES, pl.ds(0, _TAB)], tab_v, sem_t)
    cp_i = pltpu.async_copy(
        in_hbm.at[pl.ds(g * 8, 8), pl.ds(b * 128, 128)], ids_v, sem_i
    )
    cp_t.wait()
    cp_i.wait()

    def step(t, carry):
        s_hi = t // 8
        s16 = t % 8
        ids16 = ids_v[s_hi, pl.ds(s16 * _L, _L)]
        off = s_hi * 512 + s16 * _L
        for d in range(_DIM):
            col = plsc.load_gather(tab_v, [ids16 + _VOCAB * d])
            out_v[pl.ds(off + d * 128, _L)] = plsc.bitcast(col, jnp.float32)
        return carry

    lax.fori_loop(0, _STEPS, step, 0, unroll=4)
    pltpu.sync_copy(out_v, out_hbm.at[pl.ds(wid * _OPW, _OPW)])


def kernel(input_ids, embed_weight):
    ids = input_ids.reshape(_BATCH, _TILES, 128).transpose(1, 0, 2)
    ids = ids.reshape(_TILES, _BATCH * 128)
    tab = lax.bitcast_convert_type(embed_weight.T.reshape(-1), jnp.int32)
    tab_row = jnp.pad(tab, (0, _BATCH * 128 - _TAB)).reshape(1, _BATCH * 128)
    out = _embed_sc(jnp.concatenate([ids, tab_row], axis=0))
    out = out.reshape(_BATCH, _TILES, _DIM, 128).transpose(0, 1, 3, 2)
    return out.reshape(_BATCH, _SEQ, _DIM)
